# baseline (speedup 1.0000x reference)
import jax
import jax.numpy as jnp
from jax.experimental import pallas as pl
from jax.experimental.pallas import tpu as pltpu

_DIM = 384
_DQ = 16
_K = 128
_H = 56
_W = 56
_HW = _H * _W
_B = 4


def _mm(a, b):
    return jax.lax.dot_general(a, b, (((1,), (0,)), ((), ())),
                               preferred_element_type=jnp.float32)


def _vq_body(f_ref, q_ref, w_ref, sel_ref, code_ref, closs_ref, a_ref,
             wb_ref, wf_ref, w2_ref, cc_ref):
    g = pl.program_id(0)

    @pl.when(g == 0)
    def _cast_w():
        wb = w_ref[...].astype(jnp.bfloat16)
        wb_ref[...] = wb
        wf_ref[...] = wb.astype(jnp.float32)
        w2_ref[:, :_DIM] = wb
        w2_ref[:, _DIM:] = wb

    Wb = wb_ref[...]
    Wf = wf_ref[...]

    part = 0.0
    for j in range(2):
        Q = q_ref[j]
        Q1 = Q.astype(jnp.bfloat16)
        Q2 = (Q - Q1.astype(jnp.float32)).astype(jnp.bfloat16)
        a_ref[j, pl.ds(0, _DIM), :] = f_ref[j].astype(jnp.bfloat16)
        a_ref[j, pl.ds(_DIM, _DQ), :] = Q1
        a_ref[j, pl.ds(_DIM + _DQ, _DQ), :] = Q2

        M = jax.lax.dot_general(a_ref[j], a_ref[j, pl.ds(0, _DIM), :],
                                (((1,), (1,)), ((), ())),
                                preferred_element_type=jnp.float32)
        C = M[:_DIM]
        G = M[_DIM:_DIM + _DQ] + M[_DIM + _DQ:]

        C1 = C.astype(jnp.bfloat16)
        cc_ref[j, :_DIM] = C1
        cc_ref[j, _DIM:] = (C - C1.astype(jnp.float32)).astype(jnp.bfloat16)
        WC = _mm(w2_ref[...], cc_ref[j])
        TS = jnp.sum((WC.reshape(_K, _DQ, _DIM) - 2.0 * G[None])
                     * Wf.reshape(_K, _DQ, _DIM), axis=(1, 2))
        qs = jnp.sum(Q * Q)
        dist2 = TS + qs
        code = jnp.argmin(dist2).astype(jnp.int32)

        Wsel = wb_ref[pl.ds(code * _DQ, _DQ), :]
        sel = _mm(Wsel, a_ref[j, pl.ds(0, _DIM), :])
        sel_ref[j] = sel
        code_ref[j] = code.reshape(1, 1)
        part = part + jnp.sum((sel - Q) ** 2) / (_B * _DQ * _HW)

    @pl.when(g == 0)
    def _init():
        closs_ref[...] = part.reshape(1, 1)

    @pl.when(g != 0)
    def _acc():
        closs_ref[...] += part.reshape(1, 1)


def kernel(features, query, W):
    f3 = features.reshape(_B, _DIM, _HW)
    q3 = query.reshape(_B, _DQ, _HW)
    wf = W.reshape(_K * _DQ, _DIM)

    sel, codes, closs = pl.pallas_call(
        _vq_body,
        grid=(_B // 2,),
        in_specs=[
            pl.BlockSpec((2, _DIM, _HW), lambda b: (b, 0, 0)),
            pl.BlockSpec((2, _DQ, _HW), lambda b: (b, 0, 0)),
            pl.BlockSpec((_K * _DQ, _DIM), lambda b: (0, 0)),
        ],
        out_specs=[
            pl.BlockSpec((2, _DQ, _HW), lambda b: (b, 0, 0)),
            pl.BlockSpec((2, 1, 1), lambda b: (b, 0, 0)),
            pl.BlockSpec((1, 1), lambda b: (0, 0)),
        ],
        out_shape=[
            jax.ShapeDtypeStruct((_B, _DQ, _HW), jnp.float32),
            jax.ShapeDtypeStruct((_B, 1, 1), jnp.int32),
            jax.ShapeDtypeStruct((1, 1), jnp.float32),
        ],
        scratch_shapes=[
            pltpu.VMEM((2, _DIM + 2 * _DQ, _HW), jnp.bfloat16),
            pltpu.VMEM((_K * _DQ, _DIM), jnp.bfloat16),
            pltpu.VMEM((_K * _DQ, _DIM), jnp.float32),
            pltpu.VMEM((_K * _DQ, 2 * _DIM), jnp.bfloat16),
            pltpu.VMEM((2, 2 * _DIM, _DIM), jnp.bfloat16),
        ],
        compiler_params=pltpu.CompilerParams(
            dimension_semantics=("arbitrary",),
        ),
    )(f3, q3, wf)

    sel_key_values = sel.reshape(_B, _DQ, _H, _W)
    return (sel_key_values, codes.reshape(_B), closs.reshape(()))


# speedup vs baseline: 1.0106x; 1.0106x over previous
import jax
import jax.numpy as jnp
from jax.experimental import pallas as pl
from jax.experimental.pallas import tpu as pltpu

_DIM = 384
_DQ = 16
_K = 128
_H = 56
_W = 56
_HW = _H * _W
_B = 4


def _mm(a, b):
    return jax.lax.dot_general(a, b, (((1,), (0,)), ((), ())),
                               preferred_element_type=jnp.float32)


def _vq_body(f_ref, q_ref, w_ref, sel_ref, code_ref, closs_ref, a_ref,
             wb_ref, wf_ref, w2_ref, cc_ref):
    b = pl.program_id(0)

    @pl.when(b == 0)
    def _cast_w():
        wb = w_ref[...].astype(jnp.bfloat16)
        wb_ref[...] = wb
        wf_ref[...] = wb.astype(jnp.float32)
        w2_ref[:, :_DIM] = wb
        w2_ref[:, _DIM:] = wb

    Wb = wb_ref[...]
    Wf = wf_ref[...]

    Q = q_ref[0]
    Q1 = Q.astype(jnp.bfloat16)
    Q2 = (Q - Q1.astype(jnp.float32)).astype(jnp.bfloat16)
    a_ref[pl.ds(0, _DIM), :] = f_ref[0].astype(jnp.bfloat16)
    a_ref[pl.ds(_DIM, _DQ), :] = Q1
    a_ref[pl.ds(_DIM + _DQ, _DQ), :] = Q2

    M = jax.lax.dot_general(a_ref[...], a_ref[pl.ds(0, _DIM), :],
                            (((1,), (1,)), ((), ())),
                            preferred_element_type=jnp.float32)
    C = M[:_DIM]
    G = M[_DIM:_DIM + _DQ] + M[_DIM + _DQ:]

    C1 = C.astype(jnp.bfloat16)
    cc_ref[:_DIM] = C1
    cc_ref[_DIM:] = (C - C1.astype(jnp.float32)).astype(jnp.bfloat16)
    WC = _mm(w2_ref[...], cc_ref[...])
    TS = jnp.sum((WC.reshape(_K, _DQ, _DIM) - 2.0 * G[None])
                 * Wf.reshape(_K, _DQ, _DIM), axis=(1, 2))
    qs = jnp.sum(Q * Q)
    dist2 = TS + qs
    code = jnp.argmin(dist2).astype(jnp.int32)

    Wsel = wb_ref[pl.ds(code * _DQ, _DQ), :]
    sel = _mm(Wsel, a_ref[pl.ds(0, _DIM), :])
    sel_ref[0] = sel
    code_ref[...] = code.reshape(1, 1, 1)
    part = jnp.sum((sel - Q) ** 2) / (_B * _DQ * _HW)

    @pl.when(b == 0)
    def _init():
        closs_ref[...] = part.reshape(1, 1)

    @pl.when(b != 0)
    def _acc():
        closs_ref[...] += part.reshape(1, 1)


def kernel(features, query, W):
    f3 = features.reshape(_B, _DIM, _HW)
    q3 = query.reshape(_B, _DQ, _HW)
    wf = W.reshape(_K * _DQ, _DIM)

    sel, codes, closs = pl.pallas_call(
        _vq_body,
        grid=(_B,),
        in_specs=[
            pl.BlockSpec((1, _DIM, _HW), lambda b: (b, 0, 0)),
            pl.BlockSpec((1, _DQ, _HW), lambda b: (b, 0, 0)),
            pl.BlockSpec((_K * _DQ, _DIM), lambda b: (0, 0)),
        ],
        out_specs=[
            pl.BlockSpec((1, _DQ, _HW), lambda b: (b, 0, 0)),
            pl.BlockSpec((1, 1, 1), lambda b: (b, 0, 0)),
            pl.BlockSpec((1, 1), lambda b: (0, 0)),
        ],
        out_shape=[
            jax.ShapeDtypeStruct((_B, _DQ, _HW), jnp.float32),
            jax.ShapeDtypeStruct((_B, 1, 1), jnp.int32),
            jax.ShapeDtypeStruct((1, 1), jnp.float32),
        ],
        scratch_shapes=[
            pltpu.VMEM((_DIM + 2 * _DQ, _HW), jnp.bfloat16),
            pltpu.VMEM((_K * _DQ, _DIM), jnp.bfloat16),
            pltpu.VMEM((_K * _DQ, _DIM), jnp.float32),
            pltpu.VMEM((_K * _DQ, 2 * _DIM), jnp.bfloat16),
            pltpu.VMEM((2 * _DIM, _DIM), jnp.bfloat16),
        ],
        compiler_params=pltpu.CompilerParams(
            dimension_semantics=("arbitrary",),
        ),
    )(f3, q3, wf)

    sel_key_values = sel.reshape(_B, _DQ, _H, _W)
    return (sel_key_values, codes.reshape(_B), closs.reshape(()))
